# baseline (device time: 122987 ns/iter reference)
import jax
import jax.numpy as jnp
from jax import lax
from jax.experimental import pallas as pl
from jax.experimental.pallas import tpu as pltpu

N_DEV = 16


def kernel(x, w_mat):
    m_per, k = x.shape
    _, n_per = w_mat.shape

    x = x.astype(jnp.bfloat16)
    w_mat = w_mat.astype(jnp.bfloat16)

    def body(x_ref, w_ref, out_ref, comm_ref, send_sems, recv_sems):
        my = lax.axis_index("i")
        left = lax.rem(my + N_DEV - 1, N_DEV)
        right = lax.rem(my + 1, N_DEV)

        barrier_sem = pltpu.get_barrier_semaphore()
        for nbr in (left, right):
            pl.semaphore_signal(
                barrier_sem, inc=1,
                device_id=(nbr,), device_id_type=pl.DeviceIdType.MESH,
            )
        pl.semaphore_wait(barrier_sem, 2)

        comm_ref[0] = x_ref[...]

        w = w_ref[...]
        acc = jnp.dot(x_ref[...], w, preferred_element_type=jnp.float32)
        out_ref[pl.ds(my * m_per, m_per), :] = jnp.maximum(acc, 0.0)

        for h in range(N_DEV - 1):
            rdma = pltpu.make_async_remote_copy(
                src_ref=comm_ref.at[h],
                dst_ref=comm_ref.at[h + 1],
                send_sem=send_sems.at[h],
                recv_sem=recv_sems.at[h],
                device_id=(right,),
                device_id_type=pl.DeviceIdType.MESH,
            )
            rdma.start()
            rdma.wait()
            origin = lax.rem(my + N_DEV - 1 - h, N_DEV)
            acc = jnp.dot(comm_ref[h + 1], w, preferred_element_type=jnp.float32)
            out_ref[pl.ds(origin * m_per, m_per), :] = jnp.maximum(acc, 0.0)

    return pl.pallas_call(
        body,
        out_shape=jax.ShapeDtypeStruct((k, n_per), jnp.float32),
        in_specs=[
            pl.BlockSpec(memory_space=pltpu.VMEM),
            pl.BlockSpec(memory_space=pltpu.VMEM),
        ],
        out_specs=pl.BlockSpec(memory_space=pltpu.VMEM),
        scratch_shapes=[
            pltpu.VMEM((N_DEV, m_per, k), jnp.bfloat16),
            pltpu.SemaphoreType.DMA((N_DEV - 1,)),
            pltpu.SemaphoreType.DMA((N_DEV - 1,)),
        ],
        compiler_params=pltpu.CompilerParams(collective_id=0),
    )(x, w_mat)


# device time: 66941 ns/iter; 1.8372x vs baseline; 1.8372x over previous
import jax
import jax.numpy as jnp
from jax import lax
from jax.experimental import pallas as pl
from jax.experimental.pallas import tpu as pltpu

N_DEV = 16
N_FWD = 8
N_BWD = 7


def kernel(x, w_mat):
    m_per, k = x.shape
    _, n_per = w_mat.shape

    x = x.astype(jnp.bfloat16)
    w_mat = w_mat.astype(jnp.bfloat16)

    def body(x_ref, w_ref, out_ref, fwd_ref, bwd_ref,
             fwd_send_sems, fwd_recv_sems, bwd_send_sems, bwd_recv_sems):
        my = lax.axis_index("i")
        left = lax.rem(my + N_DEV - 1, N_DEV)
        right = lax.rem(my + 1, N_DEV)

        barrier_sem = pltpu.get_barrier_semaphore()
        for nbr in (left, right):
            pl.semaphore_signal(
                barrier_sem, inc=1,
                device_id=(nbr,), device_id_type=pl.DeviceIdType.MESH,
            )
        pl.semaphore_wait(barrier_sem, 2)

        fwd_ref[0] = x_ref[...]
        bwd_ref[0] = x_ref[...]

        def mk_fwd(h):
            return pltpu.make_async_remote_copy(
                src_ref=fwd_ref.at[h],
                dst_ref=fwd_ref.at[h + 1],
                send_sem=fwd_send_sems.at[h],
                recv_sem=fwd_recv_sems.at[h],
                device_id=(right,),
                device_id_type=pl.DeviceIdType.MESH,
            )

        def mk_bwd(h):
            return pltpu.make_async_remote_copy(
                src_ref=bwd_ref.at[h],
                dst_ref=bwd_ref.at[h + 1],
                send_sem=bwd_send_sems.at[h],
                recv_sem=bwd_recv_sems.at[h],
                device_id=(left,),
                device_id_type=pl.DeviceIdType.MESH,
            )

        fwd_rdmas = [mk_fwd(0)]
        bwd_rdmas = [mk_bwd(0)]
        fwd_rdmas[0].start()
        bwd_rdmas[0].start()

        w = w_ref[...]

        def compute(chunk, origin):
            acc = jnp.dot(chunk, w, preferred_element_type=jnp.float32)
            out_ref[pl.ds(origin * m_per, m_per), :] = jnp.maximum(acc, 0.0)

        compute(x_ref[...], my)

        for h in range(N_FWD):
            fwd_rdmas[h].wait_recv()
            if h + 1 < N_FWD:
                r = mk_fwd(h + 1)
                r.start()
                fwd_rdmas.append(r)
            if h < N_BWD:
                bwd_rdmas[h].wait_recv()
                if h + 1 < N_BWD:
                    r = mk_bwd(h + 1)
                    r.start()
                    bwd_rdmas.append(r)
            compute(fwd_ref[h + 1], lax.rem(my + N_DEV - 1 - h, N_DEV))
            if h < N_BWD:
                compute(bwd_ref[h + 1], lax.rem(my + 1 + h, N_DEV))

        for r in fwd_rdmas + bwd_rdmas:
            r.wait_send()

    return pl.pallas_call(
        body,
        out_shape=jax.ShapeDtypeStruct((k, n_per), jnp.float32),
        in_specs=[
            pl.BlockSpec(memory_space=pltpu.VMEM),
            pl.BlockSpec(memory_space=pltpu.VMEM),
        ],
        out_specs=pl.BlockSpec(memory_space=pltpu.VMEM),
        scratch_shapes=[
            pltpu.VMEM((N_FWD + 1, m_per, k), jnp.bfloat16),
            pltpu.VMEM((N_BWD + 1, m_per, k), jnp.bfloat16),
            pltpu.SemaphoreType.DMA((N_FWD,)),
            pltpu.SemaphoreType.DMA((N_FWD,)),
            pltpu.SemaphoreType.DMA((N_BWD,)),
            pltpu.SemaphoreType.DMA((N_BWD,)),
        ],
        compiler_params=pltpu.CompilerParams(collective_id=0),
    )(x, w_mat)


# device time: 58212 ns/iter; 2.1127x vs baseline; 1.1500x over previous
import jax
import jax.numpy as jnp
from jax import lax
from jax.experimental import pallas as pl
from jax.experimental.pallas import tpu as pltpu

N_DEV = 16
N_FWD = 8
N_BWD = 7
G = 2


def kernel(x, w_mat):
    m_per, k = x.shape
    _, n_per = w_mat.shape
    m_g = m_per // G

    x = x.astype(jnp.bfloat16)
    w_mat = w_mat.astype(jnp.bfloat16)

    def body(x_ref, w_ref, out_ref, fwd_ref, bwd_ref,
             fwd_send_sems, fwd_recv_sems, bwd_send_sems, bwd_recv_sems):
        my = lax.axis_index("i")
        left = lax.rem(my + N_DEV - 1, N_DEV)
        right = lax.rem(my + 1, N_DEV)

        barrier_sem = pltpu.get_barrier_semaphore()
        for nbr in (left, right):
            pl.semaphore_signal(
                barrier_sem, inc=1,
                device_id=(nbr,), device_id_type=pl.DeviceIdType.MESH,
            )
        pl.semaphore_wait(barrier_sem, 2)

        fwd_ref[0] = x_ref[...]
        bwd_ref[0] = x_ref[...]

        def mk(ref, sems_s, sems_r, h, g, dev):
            rows = pl.ds(g * m_g, m_g)
            return pltpu.make_async_remote_copy(
                src_ref=ref.at[h, rows, :],
                dst_ref=ref.at[h + 1, rows, :],
                send_sem=sems_s.at[h, g],
                recv_sem=sems_r.at[h, g],
                device_id=(dev,),
                device_id_type=pl.DeviceIdType.MESH,
            )

        def mk_fwd(h, g):
            return mk(fwd_ref, fwd_send_sems, fwd_recv_sems, h, g, right)

        def mk_bwd(h, g):
            return mk(bwd_ref, bwd_send_sems, bwd_recv_sems, h, g, left)

        fwd_rdmas = {}
        bwd_rdmas = {}
        for g in range(G):
            fwd_rdmas[(0, g)] = mk_fwd(0, g)
            fwd_rdmas[(0, g)].start()
            bwd_rdmas[(0, g)] = mk_bwd(0, g)
            bwd_rdmas[(0, g)].start()

        w = w_ref[...]

        def compute(chunk, origin):
            acc = jnp.dot(chunk, w, preferred_element_type=jnp.float32)
            out_ref[pl.ds(origin * m_per, m_per), :] = jnp.maximum(acc, 0.0)

        compute(x_ref[...], my)

        for h in range(N_FWD):
            for g in range(G):
                fwd_rdmas[(h, g)].wait_recv()
                if h + 1 < N_FWD:
                    r = mk_fwd(h + 1, g)
                    r.start()
                    fwd_rdmas[(h + 1, g)] = r
            if h < N_BWD:
                for g in range(G):
                    bwd_rdmas[(h, g)].wait_recv()
                    if h + 1 < N_BWD:
                        r = mk_bwd(h + 1, g)
                        r.start()
                        bwd_rdmas[(h + 1, g)] = r
            compute(fwd_ref[h + 1], lax.rem(my + N_DEV - 1 - h, N_DEV))
            if h < N_BWD:
                compute(bwd_ref[h + 1], lax.rem(my + 1 + h, N_DEV))

        for r in fwd_rdmas.values():
            r.wait_send()
        for r in bwd_rdmas.values():
            r.wait_send()

    return pl.pallas_call(
        body,
        out_shape=jax.ShapeDtypeStruct((k, n_per), jnp.float32),
        in_specs=[
            pl.BlockSpec(memory_space=pltpu.VMEM),
            pl.BlockSpec(memory_space=pltpu.VMEM),
        ],
        out_specs=pl.BlockSpec(memory_space=pltpu.VMEM),
        scratch_shapes=[
            pltpu.VMEM((N_FWD + 1, m_per, k), jnp.bfloat16),
            pltpu.VMEM((N_BWD + 1, m_per, k), jnp.bfloat16),
            pltpu.SemaphoreType.DMA((N_FWD, G)),
            pltpu.SemaphoreType.DMA((N_FWD, G)),
            pltpu.SemaphoreType.DMA((N_BWD, G)),
            pltpu.SemaphoreType.DMA((N_BWD, G)),
        ],
        compiler_params=pltpu.CompilerParams(collective_id=0),
    )(x, w_mat)
